# baseline (speedup 1.0000x reference)
import jax
import jax.numpy as jnp
from jax.experimental import pallas as pl
from jax.experimental.pallas import tpu as pltpu

_B = 64
_T = 50
_H = 128
_RH = 128
_C = 10
_CW = 5
_NCH = _T // _CW


def _dot_t(a, b):
    return jax.lax.dot_general(a, b, (((1,), (1,)), ((), ())),
                               preferred_element_type=jnp.float32)


def _gru_fc_kernel(seq_hbm, w_ih_ref, w_hh_ref, brzn_ref, bhn_ref,
                   fc_w_ref, fc_b_ref, out_ref, b0, b1, s0, s1):
    bufs = (b0, b1)
    sems = (s0, s1)

    def copy(k):
        return pltpu.make_async_copy(
            seq_hbm.at[:, pl.ds(k * _CW * _H, _CW * _H)],
            bufs[k % 2], sems[k % 2])

    copy(0).start()
    copy(1).start()

    w_ih = w_ih_ref[:, :]
    w_hh = w_hh_ref[:, :]
    brzn = brzn_ref[:, :]
    bhn = bhn_ref[:, :]

    h = jnp.zeros((_B, _RH), jnp.float32)
    for k in range(_NCH):
        copy(k).wait()
        buf = bufs[k % 2]
        for j in range(_CW):
            x_t = buf[:, j * _H:(j + 1) * _H]
            g = _dot_t(x_t, w_ih) + brzn
            gh = _dot_t(h, w_hh)
            r = 0.5 + 0.5 * jnp.tanh(0.5 * (g[:, :_RH] + gh[:, :_RH]))
            z = 0.5 + 0.5 * jnp.tanh(
                0.5 * (g[:, _RH:2 * _RH] + gh[:, _RH:2 * _RH]))
            n = jnp.tanh(g[:, 2 * _RH:] + r * (gh[:, 2 * _RH:] + bhn))
            h = n + z * (h - n)
        if k + 2 < _NCH:
            copy(k + 2).start()

    out_ref[:, :] = _dot_t(h, fc_w_ref[:, :]) + fc_b_ref[:, :]


def kernel(x, edge_index, sequences, W1, b1, W2, b2,
           w_ih, w_hh, b_ih, b_hh, fc_W, fc_b):
    seqflat = sequences.reshape(_B, _T * _H)
    brzn = (b_ih + jnp.concatenate(
        [b_hh[:2 * _RH], jnp.zeros((_RH,), jnp.float32)])).reshape(1, -1)
    bhn = b_hh[2 * _RH:].reshape(1, -1)
    vmem = pl.BlockSpec(memory_space=pltpu.MemorySpace.VMEM)
    return pl.pallas_call(
        _gru_fc_kernel,
        in_specs=[pl.BlockSpec(memory_space=pltpu.MemorySpace.HBM),
                  vmem, vmem, vmem, vmem, vmem, vmem],
        out_shape=jax.ShapeDtypeStruct((_B, _C), jnp.float32),
        scratch_shapes=[
            pltpu.VMEM((_B, _CW * _H), jnp.float32),
            pltpu.VMEM((_B, _CW * _H), jnp.float32),
            pltpu.SemaphoreType.DMA,
            pltpu.SemaphoreType.DMA,
        ],
    )(seqflat, w_ih, w_hh, brzn, bhn, fc_W, fc_b.reshape(1, -1))


# speedup vs baseline: 1.0892x; 1.0892x over previous
import jax
import jax.numpy as jnp
from jax.experimental import pallas as pl

_B = 64
_T = 50
_H = 128
_RH = 128
_C = 10


def _dot_t(a, b):
    return jax.lax.dot_general(a, b, (((1,), (1,)), ((), ())),
                               preferred_element_type=jnp.float32)


def _gru_fc_kernel(seq_ref, w_ih_ref, w_hh_ref, b_ih_ref, b_hh_ref,
                   fc_w_ref, fc_b_ref, out_ref):
    w_ih = w_ih_ref[:, :]
    w_hh = w_hh_ref[:, :]
    lane = jax.lax.broadcasted_iota(jnp.int32, (1, 3 * _RH), 1)
    brzn = b_ih_ref[:, :] + jnp.where(lane < 2 * _RH, b_hh_ref[:, :], 0.0)
    bhn = b_hh_ref[:, 2 * _RH:]

    h = jnp.zeros((_B, _RH), jnp.float32)
    for t in range(_T):
        x_t = seq_ref[:, t * _H:(t + 1) * _H]
        g = _dot_t(x_t, w_ih) + brzn
        gh = _dot_t(h, w_hh)
        r = 0.5 + 0.5 * jnp.tanh(0.5 * (g[:, :_RH] + gh[:, :_RH]))
        z = 0.5 + 0.5 * jnp.tanh(0.5 * (g[:, _RH:2 * _RH] + gh[:, _RH:2 * _RH]))
        n = jnp.tanh(g[:, 2 * _RH:] + r * (gh[:, 2 * _RH:] + bhn))
        h = n + z * (h - n)

    out_ref[:, :] = _dot_t(h, fc_w_ref[:, :]) + fc_b_ref[:, :]


def kernel(x, edge_index, sequences, W1, b1, W2, b2,
           w_ih, w_hh, b_ih, b_hh, fc_W, fc_b):
    seqflat = sequences.reshape(_B, _T * _H)
    return pl.pallas_call(
        _gru_fc_kernel,
        out_shape=jax.ShapeDtypeStruct((_B, _C), jnp.float32),
    )(seqflat, w_ih, w_hh, b_ih.reshape(1, -1), b_hh.reshape(1, -1),
      fc_W, fc_b.reshape(1, -1))
